# baseline (speedup 1.0000x reference)
import functools

import jax
import jax.numpy as jnp
from jax import lax
from jax.experimental import pallas as pl
from jax.experimental.pallas import tpu as pltpu
from jax.experimental.pallas import tpu_sc as plsc

INPUTSIZE = 2048
OUTPUTSIZE = 1024
BATCH = 2048
NCONN = 16
KTOT = 32

NW = 32
O_PER_W = OUTPUTSIZE // NW
WCHUNK = O_PER_W * INPUTSIZE


def _sc_build_wt(packed):
    mesh = plsc.VectorSubcoreMesh(core_axis_name="c", subcore_axis_name="s")

    @functools.partial(
        pl.kernel,
        mesh=mesh,
        out_type=jax.ShapeDtypeStruct((NW, O_PER_W, INPUTSIZE), jnp.float32),
        scratch_types=[
            pltpu.VMEM((O_PER_W, KTOT + NCONN), jnp.int32),
            pltpu.VMEM((O_PER_W, INPUTSIZE), jnp.float32),
            pltpu.SemaphoreType.DMA,
        ],
        compiler_params=pltpu.CompilerParams(needs_layout_passes=False),
    )
    def k(packed_hbm, wt_hbm, pk_v, wchunk, sem):
        wid = lax.axis_index("s") * 2 + lax.axis_index("c")
        obase = wid * O_PER_W
        cp_c = pltpu.async_copy(packed_hbm.at[pl.ds(obase, O_PER_W), :], pk_v, sem)

        zero = jnp.zeros((16,), jnp.float32)

        @plsc.parallel_loop(0, WCHUNK, step=16, unroll=8)
        def _zero(i):
            r = lax.shift_right_logical(i, 11)
            c = i - lax.shift_left(r, 11)
            wchunk[r, pl.ds(c, 16)] = zero

        cp_c.wait()

        lane = lax.iota(jnp.int32, 16)
        one = jnp.ones((16,), jnp.float32)
        for g in range(O_PER_W // 16):
            olocal = g * 16 + lane
            plsc.addupdate_scatter(wchunk, [olocal, obase + olocal], one)

            def sbody(kk, carry, olocal=olocal):
                kvec = jnp.full((16,), kk, jnp.int32)
                wvec = jnp.full((16,), KTOT, jnp.int32) + (kvec & (NCONN - 1))
                cv = plsc.load_gather(pk_v, [olocal, kvec])
                wv = plsc.bitcast(plsc.load_gather(pk_v, [olocal, wvec]), jnp.float32)
                plsc.addupdate_scatter(wchunk, [olocal, cv], wv)
                return carry

            lax.fori_loop(0, KTOT, sbody, 0)

        pltpu.sync_copy(wchunk, wt_hbm.at[wid])

    return k(packed)


_BM = 1024


def _mm_body(xb_ref, wt_ref, b_ref, o_ref):
    acc = lax.dot_general(
        xb_ref[...], wt_ref[...],
        (((1,), (1,)), ((), ())),
        preferred_element_type=jnp.float32,
    )
    o_ref[...] = acc + b_ref[...]


def _mm(xb, wt, bias2d):
    return pl.pallas_call(
        _mm_body,
        grid=(BATCH // _BM,),
        in_specs=[
            pl.BlockSpec((_BM, INPUTSIZE), lambda i: (i, 0)),
            pl.BlockSpec((OUTPUTSIZE, INPUTSIZE), lambda i: (0, 0)),
            pl.BlockSpec((1, OUTPUTSIZE), lambda i: (0, 0)),
        ],
        out_specs=pl.BlockSpec((_BM, OUTPUTSIZE), lambda i: (i, 0)),
        out_shape=jax.ShapeDtypeStruct((BATCH, OUTPUTSIZE), jnp.float32),
    )(xb, wt, bias2d)


def kernel(x, weights, bias, conns):
    xb = x.astype(jnp.bfloat16)
    packed = jnp.concatenate(
        [conns, lax.bitcast_convert_type(weights, jnp.int32)], axis=1
    )
    wt = _sc_build_wt(packed).reshape(OUTPUTSIZE, INPUTSIZE)
    return _mm(xb, wt, bias.reshape(1, OUTPUTSIZE))


# speedup vs baseline: 1.0145x; 1.0145x over previous
import functools

import jax
import jax.numpy as jnp
from jax import lax
from jax.experimental import pallas as pl
from jax.experimental.pallas import tpu as pltpu
from jax.experimental.pallas import tpu_sc as plsc

INPUTSIZE = 2048
OUTPUTSIZE = 1024
BATCH = 2048
NCONN = 16
KTOT = 32

NW = 32
O_PER_W = OUTPUTSIZE // NW
WCHUNK = O_PER_W * INPUTSIZE


def _sc_build_wt(packed):
    mesh = plsc.VectorSubcoreMesh(core_axis_name="c", subcore_axis_name="s")

    @functools.partial(
        pl.kernel,
        mesh=mesh,
        out_type=jax.ShapeDtypeStruct((NW, O_PER_W, INPUTSIZE), jnp.float32),
        scratch_types=[
            pltpu.VMEM((O_PER_W, KTOT + NCONN), jnp.int32),
            pltpu.VMEM((O_PER_W, INPUTSIZE), jnp.float32),
            pltpu.SemaphoreType.DMA,
        ],
        compiler_params=pltpu.CompilerParams(needs_layout_passes=False),
    )
    def k(packed_hbm, wt_hbm, pk_v, wchunk, sem):
        wid = lax.axis_index("s") * 2 + lax.axis_index("c")
        obase = wid * O_PER_W
        cp_c = pltpu.async_copy(packed_hbm.at[pl.ds(obase, O_PER_W), :], pk_v, sem)

        zero = jnp.zeros((16,), jnp.float32)

        @plsc.parallel_loop(0, WCHUNK, step=16, unroll=8)
        def _zero(i):
            r = lax.shift_right_logical(i, 11)
            c = i - lax.shift_left(r, 11)
            wchunk[r, pl.ds(c, 16)] = zero

        cp_c.wait()

        lane = lax.iota(jnp.int32, 16)
        one = jnp.ones((16,), jnp.float32)
        for g in range(O_PER_W // 16):
            olocal = g * 16 + lane
            plsc.addupdate_scatter(wchunk, [olocal, obase + olocal], one)

            def sbody(kk, carry, olocal=olocal):
                kvec = jnp.full((16,), kk, jnp.int32)
                wvec = jnp.full((16,), KTOT, jnp.int32) + (kvec & (NCONN - 1))
                cv = plsc.load_gather(pk_v, [olocal, kvec])
                wv = plsc.bitcast(plsc.load_gather(pk_v, [olocal, wvec]), jnp.float32)
                plsc.addupdate_scatter(wchunk, [olocal, cv], wv)
                return carry

            lax.fori_loop(0, KTOT, sbody, 0)

        pltpu.sync_copy(wchunk, wt_hbm.at[wid])

    return k(packed)


_BM = 512


def _mm_body(xb_ref, wt_ref, b_ref, o_ref):
    acc = lax.dot_general(
        xb_ref[...], wt_ref[...],
        (((1,), (1,)), ((), ())),
        preferred_element_type=jnp.float32,
    )
    o_ref[...] = acc + b_ref[...]


def _mm(xb, wt, bias2d):
    return pl.pallas_call(
        _mm_body,
        grid=(BATCH // _BM,),
        in_specs=[
            pl.BlockSpec((_BM, INPUTSIZE), lambda i: (i, 0)),
            pl.BlockSpec((OUTPUTSIZE, INPUTSIZE), lambda i: (0, 0)),
            pl.BlockSpec((1, OUTPUTSIZE), lambda i: (0, 0)),
        ],
        out_specs=pl.BlockSpec((_BM, OUTPUTSIZE), lambda i: (i, 0)),
        out_shape=jax.ShapeDtypeStruct((BATCH, OUTPUTSIZE), jnp.float32),
    )(xb, wt, bias2d)


def kernel(x, weights, bias, conns):
    xb = x.astype(jnp.bfloat16)
    packed = jnp.concatenate(
        [conns, lax.bitcast_convert_type(weights, jnp.int32)], axis=1
    )
    wt = _sc_build_wt(packed).reshape(OUTPUTSIZE, INPUTSIZE)
    return _mm(xb, wt, bias.reshape(1, OUTPUTSIZE))
